# baseline (speedup 1.0000x reference)
import functools

import jax
import jax.numpy as jnp
from jax.experimental import pallas as pl
from jax.experimental.pallas import tpu as pltpu

BLK = 1024
NBUF = 3


def _route_chunk(x, w, b):
    P = w.shape[0]
    logits_t = jax.lax.dot_general(
        w, x, (((1,), (1,)), ((), ())),
        preferred_element_type=jnp.float32)
    logits_t = logits_t + b
    m = jnp.max(logits_t, axis=0, keepdims=True)
    sub = jax.lax.broadcasted_iota(jnp.int32, logits_t.shape, 0)
    sel = jnp.min(jnp.where(logits_t == m, sub, P), axis=0)
    sel = sel.astype(jnp.int32)
    mask_t = (sub == sel[None, :]).astype(jnp.float32)
    return sel, mask_t.T


def _router_kernel(x1_ref, x2_hbm, w_ref, b_ref, sel_hbm, mask_hbm,
                   xbuf2, selbuf, maskbuf, in_sems, sel_sems, mask_sems):
    i = pl.program_id(0)
    n = pl.num_programs(0)
    T2 = x2_hbm.shape[0] // 2

    def in_copy(c, slot):
        return pltpu.make_async_copy(
            x2_hbm.at[pl.ds(T2 + c * BLK, BLK), :], xbuf2.at[slot],
            in_sems.at[slot])

    def mask_copy(c, slot, h):
        base = jnp.where(h == 0, c * BLK, T2 + c * BLK)
        return pltpu.make_async_copy(
            maskbuf.at[slot, h], mask_hbm.at[pl.ds(base, BLK), :],
            mask_sems.at[slot, h])

    def sel_copy(c, slot, h):
        base = jnp.where(h == 0, c * BLK, T2 + c * BLK)
        return pltpu.make_async_copy(
            selbuf.at[slot, h], sel_hbm.at[:, pl.ds(base, BLK)],
            sel_sems.at[slot, h])

    @pl.when(i == 0)
    def _():
        in_copy(0, 0).start()
        in_copy(1, 1).start()

    slot = jax.lax.rem(i, NBUF)
    oslot = jax.lax.rem(i, 2)
    in_copy(i, slot).wait()

    w = w_ref[...]
    b = b_ref[...]
    sel1, mask1 = _route_chunk(x1_ref[...], w, b)
    sel2, mask2 = _route_chunk(xbuf2[slot], w, b)

    @pl.when(i >= 2)
    def _():
        for h in (0, 1):
            mask_copy(i - 2, oslot, h).wait()
            sel_copy(i - 2, oslot, h).wait()

    maskbuf[oslot, 0] = mask1
    maskbuf[oslot, 1] = mask2
    selbuf[oslot, 0, 0, :] = sel1
    selbuf[oslot, 1, 0, :] = sel2
    for h in (0, 1):
        mask_copy(i, oslot, h).start()
        sel_copy(i, oslot, h).start()

    @pl.when(i + 2 < n)
    def _():
        in_copy(i + 2, jax.lax.rem(i + 2, NBUF)).start()

    @pl.when(i == n - 1)
    def _():
        for di in (1, 0):
            c = n - 1 - di
            osl = jax.lax.rem(c, 2)
            for h in (0, 1):
                mask_copy(c, osl, h).wait()
                sel_copy(c, osl, h).wait()


@functools.partial(jax.jit, static_argnames=())
def kernel(process_feats, routing_matrix, bias):
    B, N, D = process_feats.shape
    P = routing_matrix.shape[0]
    T = B * N
    x = process_feats.reshape(T, D)
    b = bias.reshape(P, 1)
    grid = (T // 2 // BLK,)
    sel2d, mask = pl.pallas_call(
        _router_kernel,
        grid=grid,
        in_specs=[
            pl.BlockSpec((BLK, D), lambda i: (i, 0)),
            pl.BlockSpec(memory_space=pltpu.MemorySpace.HBM),
            pl.BlockSpec((P, D), lambda i: (0, 0)),
            pl.BlockSpec((P, 1), lambda i: (0, 0)),
        ],
        out_specs=[
            pl.BlockSpec(memory_space=pltpu.MemorySpace.HBM),
            pl.BlockSpec(memory_space=pltpu.MemorySpace.HBM),
        ],
        out_shape=[
            jax.ShapeDtypeStruct((1, T), jnp.int32),
            jax.ShapeDtypeStruct((T, P), jnp.float32),
        ],
        scratch_shapes=[
            pltpu.VMEM((NBUF, BLK, D), jnp.float32),
            pltpu.VMEM((2, 2, 1, BLK), jnp.int32),
            pltpu.VMEM((2, 2, BLK, P), jnp.float32),
            pltpu.SemaphoreType.DMA((NBUF,)),
            pltpu.SemaphoreType.DMA((2, 2)),
            pltpu.SemaphoreType.DMA((2, 2)),
        ],
        compiler_params=pltpu.CompilerParams(
            dimension_semantics=("arbitrary",),
        ),
    )(x, x, routing_matrix, b)
    selected = sel2d.reshape(B, N)
    policy_mask = mask.reshape(B, N, P)
    return (selected, policy_mask)


# speedup vs baseline: 1.0439x; 1.0439x over previous
import functools

import jax
import jax.numpy as jnp
from jax.experimental import pallas as pl
from jax.experimental.pallas import tpu as pltpu

BLK = 2048


def _router_kernel(x_ref, w_ref, b_ref, sel_ref, mask_ref):
    x = x_ref[...]
    w = w_ref[...]
    P = w.shape[0]
    logits_t = jax.lax.dot_general(
        w, x, (((1,), (1,)), ((), ())),
        preferred_element_type=jnp.float32)
    logits_t = logits_t + b_ref[...]
    m = jnp.max(logits_t, axis=0, keepdims=True)
    sub = jax.lax.broadcasted_iota(jnp.int32, logits_t.shape, 0)
    sel = jnp.min(jnp.where(logits_t == m, sub, P), axis=0)
    sel = sel.astype(jnp.int32)
    mask_t = (sub == sel[None, :]).astype(jnp.float32)
    mask_ref[...] = mask_t.T
    sel_ref[0, 0, :] = sel


@functools.partial(jax.jit, static_argnames=())
def kernel(process_feats, routing_matrix, bias):
    B, N, D = process_feats.shape
    P = routing_matrix.shape[0]
    T = B * N
    x = process_feats.reshape(T, D)
    b = bias.reshape(P, 1)
    grid = (T // BLK,)
    sel2d, mask = pl.pallas_call(
        _router_kernel,
        grid=grid,
        in_specs=[
            pl.BlockSpec((BLK, D), lambda i: (i, 0)),
            pl.BlockSpec((P, D), lambda i: (0, 0)),
            pl.BlockSpec((P, 1), lambda i: (0, 0)),
        ],
        out_specs=[
            pl.BlockSpec((1, 1, BLK), lambda i: (i, 0, 0)),
            pl.BlockSpec((BLK, P), lambda i: (i, 0)),
        ],
        out_shape=[
            jax.ShapeDtypeStruct((T // BLK, 1, BLK), jnp.int32),
            jax.ShapeDtypeStruct((T, P), jnp.float32),
        ],
        compiler_params=pltpu.CompilerParams(
            dimension_semantics=("arbitrary",),
        ),
    )(x, routing_matrix, b)
    selected = sel2d.reshape(B, N)
    policy_mask = mask.reshape(B, N, P)
    return (selected, policy_mask)
